# baseline (speedup 1.0000x reference)
import functools

import jax
import jax.numpy as jnp
from jax import lax
from jax.experimental import pallas as pl
from jax.experimental.pallas import tpu as pltpu
from jax.experimental.pallas import tpu_sc as plsc

B = 4096
NUM_NUMERIC = 13
NUM_FIELDS = 26
VOCAB = 100000
EMB = 32
EPS = 1e-5

NC = 2
NS = 16
NW = NC * NS
TOT = B * NUM_FIELDS
ROWS_PER_W = TOT // NW
CHUNK = 128
NCHUNK = ROWS_PER_W // CHUNK


def _gather_body(table_hbm, idx_hbm, out_hbm, idx_v, rows_v, sem):
    wid = lax.axis_index("s") * NC + lax.axis_index("c")
    pltpu.sync_copy(idx_hbm.at[wid], idx_v)
    copies = []
    for j in range(NCHUNK):
        copies.append(
            pltpu.async_copy(table_hbm.at[idx_v.at[j]], rows_v.at[j], sem)
        )
    for c in copies:
        c.wait()
    pltpu.sync_copy(rows_v, out_hbm.at[wid])


@functools.cache
def _make_gather():
    return pl.kernel(
        _gather_body,
        out_type=jax.ShapeDtypeStruct((NW, NCHUNK, CHUNK, EMB), jnp.float32),
        mesh=plsc.VectorSubcoreMesh(core_axis_name="c", subcore_axis_name="s",
                                    num_cores=NC, num_subcores=NS),
        scratch_types=[
            pltpu.VMEM((NCHUNK, CHUNK), jnp.int32),
            pltpu.VMEM((NCHUNK, CHUNK, EMB), jnp.float32),
            pltpu.SemaphoreType.DMA,
        ],
    )


def _bn_relu(h, g, be):
    h = jnp.maximum(h, 0.0)
    mean = jnp.mean(h, axis=0, keepdims=True)
    c = h - mean
    var = jnp.mean(c * c, axis=0, keepdims=True)
    return c * (g * lax.rsqrt(var + EPS)) + be


def _mlp_body(emb_ref, num_ref, w0e_ref, w0n_ref, b0_ref, g0_ref, be0_ref,
              w1_ref, b1_ref, g1_ref, be1_ref,
              w2_ref, b2_ref, g2_ref, be2_ref,
              wh_ref, bh_ref, out_ref):
    dn = (((1,), (1,)), ((), ()))
    h = lax.dot_general(emb_ref[...], w0e_ref[...], dn,
                        preferred_element_type=jnp.float32)
    h = h + lax.dot_general(num_ref[...], w0n_ref[...], dn,
                            preferred_element_type=jnp.float32)
    h = _bn_relu(h + b0_ref[...], g0_ref[...], be0_ref[...])
    h = lax.dot_general(h, w1_ref[...], dn, preferred_element_type=jnp.float32)
    h = _bn_relu(h + b1_ref[...], g1_ref[...], be1_ref[...])
    h = lax.dot_general(h, w2_ref[...], dn, preferred_element_type=jnp.float32)
    h = _bn_relu(h + b2_ref[...], g2_ref[...], be2_ref[...])
    out = lax.dot_general(h, wh_ref[...], dn,
                          preferred_element_type=jnp.float32)
    out_ref[...] = out + bh_ref[...]


def kernel(numeric, categorical, tables,
           W0, b0, g0, be0, W1, b1, g1, be1, W2, b2, g2, be2, Wh, bh):
    flat_tables = tables.reshape(NUM_FIELDS * VOCAB, EMB)
    offs = (jnp.arange(NUM_FIELDS, dtype=jnp.int32) * VOCAB)[None, :]
    flat_idx = (categorical + offs).reshape(-1)
    embeds = jnp.take(flat_tables, flat_idx, axis=0,
                      mode="clip").astype(jnp.bfloat16)
    embeds = embeds.reshape(B, NUM_FIELDS * EMB)

    W0n = W0[:, :NUM_NUMERIC]
    W0e = W0[:, NUM_NUMERIC:].astype(jnp.bfloat16)
    Wh128 = jnp.zeros((128, Wh.shape[1]), Wh.dtype).at[0].set(Wh[0])
    bh128 = jnp.zeros((1, 128), bh.dtype).at[0, 0].set(bh[0])
    out = pl.pallas_call(
        _mlp_body,
        out_shape=jax.ShapeDtypeStruct((B, 128), jnp.float32),
    )(embeds, numeric, W0e, W0n,
      b0.reshape(1, -1), g0.reshape(1, -1), be0.reshape(1, -1),
      W1, b1.reshape(1, -1), g1.reshape(1, -1), be1.reshape(1, -1),
      W2, b2.reshape(1, -1), g2.reshape(1, -1), be2.reshape(1, -1),
      Wh128, bh128)
    return out[:, 0]


# speedup vs baseline: 7.9539x; 7.9539x over previous
import functools

import jax
import jax.numpy as jnp
from jax import lax
from jax.experimental import pallas as pl
from jax.experimental.pallas import tpu as pltpu
from jax.experimental.pallas import tpu_sc as plsc

B = 4096
NUM_NUMERIC = 13
NUM_FIELDS = 26
VOCAB = 100000
EMB = 32
EPS = 1e-5

NC = 2
NS = 16
NW = NC * NS
TOT = B * NUM_FIELDS
ROWS_PER_W = TOT // NW
CHUNK = 128
NCHUNK = ROWS_PER_W // CHUNK


def _gather_body(table_hbm, idx_hbm, out_hbm, idx_v, rows_v, sem):
    wid = lax.axis_index("s") * NC + lax.axis_index("c")
    pltpu.sync_copy(idx_hbm.at[wid], idx_v)
    copies = []
    for j in range(NCHUNK):
        copies.append(
            pltpu.async_copy(table_hbm.at[idx_v.at[j]], rows_v.at[j], sem)
        )
    for c in copies:
        c.wait()
    pltpu.sync_copy(rows_v, out_hbm.at[wid])


@functools.cache
def _make_gather():
    return pl.kernel(
        _gather_body,
        out_type=jax.ShapeDtypeStruct((NW, NCHUNK, CHUNK, EMB), jnp.float32),
        mesh=plsc.VectorSubcoreMesh(core_axis_name="c", subcore_axis_name="s",
                                    num_cores=NC, num_subcores=NS),
        scratch_types=[
            pltpu.VMEM((NCHUNK, CHUNK), jnp.int32),
            pltpu.VMEM((NCHUNK, CHUNK, EMB), jnp.float32),
            pltpu.SemaphoreType.DMA,
        ],
        compiler_params=pltpu.CompilerParams(use_tc_tiling_on_sc=False),
    )


def _bn_relu(h, g, be):
    h = jnp.maximum(h, 0.0)
    mean = jnp.mean(h, axis=0, keepdims=True)
    c = h - mean
    var = jnp.mean(c * c, axis=0, keepdims=True)
    return c * (g * lax.rsqrt(var + EPS)) + be


def _mlp_body(emb_ref, num_ref, w0e_ref, w0n_ref, b0_ref, g0_ref, be0_ref,
              w1_ref, b1_ref, g1_ref, be1_ref,
              w2_ref, b2_ref, g2_ref, be2_ref,
              wh_ref, bh_ref, out_ref):
    dn = (((1,), (1,)), ((), ()))
    h = lax.dot_general(emb_ref[...], w0e_ref[...], dn,
                        preferred_element_type=jnp.float32)
    h = h + lax.dot_general(num_ref[...], w0n_ref[...], dn,
                            preferred_element_type=jnp.float32)
    h = _bn_relu(h + b0_ref[...], g0_ref[...], be0_ref[...])
    h = lax.dot_general(h, w1_ref[...], dn, preferred_element_type=jnp.float32)
    h = _bn_relu(h + b1_ref[...], g1_ref[...], be1_ref[...])
    h = lax.dot_general(h, w2_ref[...], dn, preferred_element_type=jnp.float32)
    h = _bn_relu(h + b2_ref[...], g2_ref[...], be2_ref[...])
    out = lax.dot_general(h, wh_ref[...], dn,
                          preferred_element_type=jnp.float32)
    out_ref[...] = out + bh_ref[...]


def kernel(numeric, categorical, tables,
           W0, b0, g0, be0, W1, b1, g1, be1, W2, b2, g2, be2, Wh, bh):
    flat_tables = tables.reshape(NUM_FIELDS * VOCAB, EMB)
    offs = (jnp.arange(NUM_FIELDS, dtype=jnp.int32) * VOCAB)[None, :]
    flat_idx = (categorical + offs).reshape(NW, NCHUNK, CHUNK)
    embeds = _make_gather()(flat_tables, flat_idx).astype(jnp.bfloat16)
    embeds = embeds.reshape(B, NUM_FIELDS * EMB)

    W0n = W0[:, :NUM_NUMERIC]
    W0e = W0[:, NUM_NUMERIC:].astype(jnp.bfloat16)
    Wh128 = jnp.zeros((128, Wh.shape[1]), Wh.dtype).at[0].set(Wh[0])
    bh128 = jnp.zeros((1, 128), bh.dtype).at[0, 0].set(bh[0])
    out = pl.pallas_call(
        _mlp_body,
        out_shape=jax.ShapeDtypeStruct((B, 128), jnp.float32),
    )(embeds, numeric, W0e, W0n,
      b0.reshape(1, -1), g0.reshape(1, -1), be0.reshape(1, -1),
      W1, b1.reshape(1, -1), g1.reshape(1, -1), be1.reshape(1, -1),
      W2, b2.reshape(1, -1), g2.reshape(1, -1), be2.reshape(1, -1),
      Wh128, bh128)
    return out[:, 0]


# speedup vs baseline: 13.7254x; 1.7256x over previous
import functools

import jax
import jax.numpy as jnp
from jax import lax
from jax.experimental import pallas as pl
from jax.experimental.pallas import tpu as pltpu
from jax.experimental.pallas import tpu_sc as plsc

B = 4096
NUM_NUMERIC = 13
NUM_FIELDS = 26
VOCAB = 100000
EMB = 32
EPS = 1e-5

NC = 2
NS = 16
NW = NC * NS
TOT = B * NUM_FIELDS * EMB
PER_W = TOT // NW
NCHUNK = 4
CH = PER_W // NCHUNK


def _gather_body(flat_hbm, idx_hbm, out_hbm,
                 idx_a, idx_b, val_a, val_b, sem_a, sem_b):
    wid = lax.axis_index("s") * NC + lax.axis_index("c")
    idx_v = [idx_a, idx_b]
    val_v = [val_a, val_b]
    sem = [sem_a, sem_b]

    def stage(cb, buf):
        pltpu.sync_copy(idx_hbm.at[wid, cb], idx_v[buf])
        return pltpu.async_copy(flat_hbm.at[idx_v[buf]], val_v[buf], sem[buf])

    copies = [None, None]
    copies[0] = stage(0, 0)
    for cb in range(1, NCHUNK):
        buf = cb % 2
        copies[buf] = stage(cb, buf)
        copies[1 - buf].wait()
        pltpu.sync_copy(val_v[1 - buf], out_hbm.at[wid * NCHUNK + cb - 1])
    copies[(NCHUNK - 1) % 2].wait()
    pltpu.sync_copy(val_v[(NCHUNK - 1) % 2],
                    out_hbm.at[wid * NCHUNK + NCHUNK - 1])


@functools.cache
def _make_gather():
    return pl.kernel(
        _gather_body,
        out_type=jax.ShapeDtypeStruct((NW * NCHUNK, CH), jnp.float32),
        mesh=plsc.VectorSubcoreMesh(core_axis_name="c", subcore_axis_name="s",
                                    num_cores=NC, num_subcores=NS),
        scratch_types=[
            pltpu.VMEM((CH,), jnp.int32),
            pltpu.VMEM((CH,), jnp.int32),
            pltpu.VMEM((CH,), jnp.float32),
            pltpu.VMEM((CH,), jnp.float32),
            pltpu.SemaphoreType.DMA,
            pltpu.SemaphoreType.DMA,
        ],
    )


def _bn_relu(h, g, be):
    h = jnp.maximum(h, 0.0)
    mean = jnp.mean(h, axis=0, keepdims=True)
    c = h - mean
    var = jnp.mean(c * c, axis=0, keepdims=True)
    return c * (g * lax.rsqrt(var + EPS)) + be


def _mlp_body(emb_ref, num_ref, w0e_ref, w0n_ref, b0_ref, g0_ref, be0_ref,
              w1_ref, b1_ref, g1_ref, be1_ref,
              w2_ref, b2_ref, g2_ref, be2_ref,
              wh_ref, bh_ref, out_ref):
    dn = (((1,), (1,)), ((), ()))
    h = lax.dot_general(emb_ref[...], w0e_ref[...], dn,
                        preferred_element_type=jnp.float32)
    h = h + lax.dot_general(num_ref[...], w0n_ref[...], dn,
                            preferred_element_type=jnp.float32)
    h = _bn_relu(h + b0_ref[...], g0_ref[...], be0_ref[...])
    h = lax.dot_general(h, w1_ref[...], dn, preferred_element_type=jnp.float32)
    h = _bn_relu(h + b1_ref[...], g1_ref[...], be1_ref[...])
    h = lax.dot_general(h, w2_ref[...], dn, preferred_element_type=jnp.float32)
    h = _bn_relu(h + b2_ref[...], g2_ref[...], be2_ref[...])
    out = lax.dot_general(h, wh_ref[...], dn,
                          preferred_element_type=jnp.float32)
    out_ref[...] = out + bh_ref[...]


def kernel(numeric, categorical, tables,
           W0, b0, g0, be0, W1, b1, g1, be1, W2, b2, g2, be2, Wh, bh):
    flat = tables.transpose(0, 2, 1).reshape(-1)
    f_off = (jnp.arange(NUM_FIELDS, dtype=jnp.int32)
             * (EMB * VOCAB))[None, :, None]
    e_off = (jnp.arange(EMB, dtype=jnp.int32) * VOCAB)[None, None, :]
    offs = (categorical[:, :, None] + f_off + e_off)
    offs = offs.reshape(NW, NCHUNK, CH)
    gathered = _make_gather()(flat, offs)
    embeds = gathered.reshape(B, NUM_FIELDS * EMB).astype(jnp.bfloat16)

    W0n = W0[:, :NUM_NUMERIC]
    W0e = W0[:, NUM_NUMERIC:].astype(jnp.bfloat16)
    Wh128 = jnp.zeros((128, Wh.shape[1]), Wh.dtype).at[0].set(Wh[0])
    bh128 = jnp.zeros((1, 128), bh.dtype).at[0, 0].set(bh[0])
    out = pl.pallas_call(
        _mlp_body,
        out_shape=jax.ShapeDtypeStruct((B, 128), jnp.float32),
    )(embeds, numeric, W0e, W0n,
      b0.reshape(1, -1), g0.reshape(1, -1), be0.reshape(1, -1),
      W1, b1.reshape(1, -1), g1.reshape(1, -1), be1.reshape(1, -1),
      W2, b2.reshape(1, -1), g2.reshape(1, -1), be2.reshape(1, -1),
      Wh128, bh128)
    return out[:, 0]


# speedup vs baseline: 13.9277x; 1.0147x over previous
import functools

import jax
import jax.numpy as jnp
from jax import lax
from jax.experimental import pallas as pl
from jax.experimental.pallas import tpu as pltpu
from jax.experimental.pallas import tpu_sc as plsc

B = 4096
NUM_NUMERIC = 13
NUM_FIELDS = 26
VOCAB = 100000
EMB = 32
EPS = 1e-5

NC = 2
NS = 16
NW = NC * NS
TOT = B * NUM_FIELDS * EMB
PER_W = TOT // NW
NCHUNK = 4
CH = PER_W // NCHUNK


def _gather_body(flat_hbm, idx_hbm, out_hbm,
                 idx_a, idx_b, val_a, val_b, sem_a, sem_b):
    wid = lax.axis_index("s") * NC + lax.axis_index("c")
    idx_v = [idx_a, idx_b]
    val_v = [val_a, val_b]
    sem = [sem_a, sem_b]

    def stage(cb, buf):
        pltpu.sync_copy(idx_hbm.at[wid * NCHUNK + cb], idx_v[buf])
        return pltpu.async_copy(flat_hbm.at[idx_v[buf]], val_v[buf], sem[buf])

    copies = [None, None]
    copies[0] = stage(0, 0)
    for cb in range(1, NCHUNK):
        buf = cb % 2
        copies[buf] = stage(cb, buf)
        copies[1 - buf].wait()
        pltpu.sync_copy(val_v[1 - buf], out_hbm.at[wid * NCHUNK + cb - 1])
    copies[(NCHUNK - 1) % 2].wait()
    pltpu.sync_copy(val_v[(NCHUNK - 1) % 2],
                    out_hbm.at[wid * NCHUNK + NCHUNK - 1])


@functools.cache
def _make_gather():
    return pl.kernel(
        _gather_body,
        out_type=jax.ShapeDtypeStruct((NW * NCHUNK, CH), jnp.float32),
        mesh=plsc.VectorSubcoreMesh(core_axis_name="c", subcore_axis_name="s",
                                    num_cores=NC, num_subcores=NS),
        scratch_types=[
            pltpu.VMEM((CH,), jnp.int32),
            pltpu.VMEM((CH,), jnp.int32),
            pltpu.VMEM((CH,), jnp.float32),
            pltpu.VMEM((CH,), jnp.float32),
            pltpu.SemaphoreType.DMA,
            pltpu.SemaphoreType.DMA,
        ],
    )


def _bn_relu(h, g, be):
    h = jnp.maximum(h, 0.0)
    mean = jnp.mean(h, axis=0, keepdims=True)
    c = h - mean
    var = jnp.mean(c * c, axis=0, keepdims=True)
    return c * (g * lax.rsqrt(var + EPS)) + be


def _mlp_body(emb_ref, num_ref, w0e_ref, w0n_ref, b0_ref, g0_ref, be0_ref,
              w1_ref, b1_ref, g1_ref, be1_ref,
              w2_ref, b2_ref, g2_ref, be2_ref,
              wh_ref, bh_ref, out_ref):
    dn = (((1,), (1,)), ((), ()))
    h = lax.dot_general(emb_ref[...], w0e_ref[...], dn,
                        preferred_element_type=jnp.float32)
    h = h + lax.dot_general(num_ref[...], w0n_ref[...], dn,
                            preferred_element_type=jnp.float32)
    h = _bn_relu(h + b0_ref[...], g0_ref[...], be0_ref[...])
    h = lax.dot_general(h, w1_ref[...], dn, preferred_element_type=jnp.float32)
    h = _bn_relu(h + b1_ref[...], g1_ref[...], be1_ref[...])
    h = lax.dot_general(h, w2_ref[...], dn, preferred_element_type=jnp.float32)
    h = _bn_relu(h + b2_ref[...], g2_ref[...], be2_ref[...])
    out = lax.dot_general(h, wh_ref[...], dn,
                          preferred_element_type=jnp.float32)
    out_ref[...] = out + bh_ref[...]


def kernel(numeric, categorical, tables,
           W0, b0, g0, be0, W1, b1, g1, be1, W2, b2, g2, be2, Wh, bh):
    flat = tables.transpose(0, 2, 1).reshape(-1)
    f_off = (jnp.arange(NUM_FIELDS, dtype=jnp.int32)
             * (EMB * VOCAB))[None, :, None]
    e_off = (jnp.arange(EMB, dtype=jnp.int32) * VOCAB)[None, None, :]
    offs = (categorical[:, :, None] + f_off + e_off)
    offs = offs.reshape(NW * NCHUNK, CH)
    gathered = _make_gather()(flat, offs)
    embeds = gathered.reshape(B, NUM_FIELDS * EMB).astype(jnp.bfloat16)

    W0n = W0[:, :NUM_NUMERIC]
    W0e = W0[:, NUM_NUMERIC:].astype(jnp.bfloat16)
    Wh128 = jnp.zeros((128, Wh.shape[1]), Wh.dtype).at[0].set(Wh[0])
    bh128 = jnp.zeros((1, 128), bh.dtype).at[0, 0].set(bh[0])
    out = pl.pallas_call(
        _mlp_body,
        out_shape=jax.ShapeDtypeStruct((B, 128), jnp.float32),
    )(embeds, numeric, W0e, W0n,
      b0.reshape(1, -1), g0.reshape(1, -1), be0.reshape(1, -1),
      W1, b1.reshape(1, -1), g1.reshape(1, -1), be1.reshape(1, -1),
      W2, b2.reshape(1, -1), g2.reshape(1, -1), be2.reshape(1, -1),
      Wh128, bh128)
    return out[:, 0]
